# baseline (speedup 1.0000x reference)
import dataclasses
import functools

import jax
import jax.numpy as jnp
import numpy as np
from jax import lax
from jax.experimental import pallas as pl
from jax.experimental.pallas import tpu as pltpu
from jax.experimental.pallas import tpu_sc as plsc

N = 10000
N_PAD = 10240
E = 320000
E_PAD = 327680
D = 128
H = 128
G = 64
NC = 2
NS = 16
NW = NC * NS
L = 16
WIN = 128
NWIN = E_PAD // WIN
NI = NWIN // NW
ZCH = 80
BN = 1000
NB = N // BN
MAGIC = np.int32(0x5F3759DF)


def _mesh():
    return plsc.VectorSubcoreMesh(
        core_axis_name="c", subcore_axis_name="s", num_cores=NC, num_subcores=NS
    )


def _sc_params():
    cp = pltpu.CompilerParams()
    if "needs_layout_passes" in pltpu.CompilerParams.__dataclass_fields__:
        cp = dataclasses.replace(cp, needs_layout_passes=False)
    return cp


def _sc_preprocess(px, py, pz, src, dst):
    @functools.partial(
        pl.kernel,
        out_type=(
            jax.ShapeDtypeStruct((E_PAD,), jnp.float32),
            jax.ShapeDtypeStruct((NC, N_PAD), jnp.float32),
        ),
        mesh=_mesh(),
        scratch_types=[
            pltpu.VMEM((N,), jnp.float32),
            pltpu.VMEM((N,), jnp.float32),
            pltpu.VMEM((N,), jnp.float32),
            pltpu.VMEM((2, WIN), jnp.int32),
            pltpu.VMEM((2, WIN), jnp.int32),
            pltpu.VMEM((2, WIN), jnp.float32),
            pltpu.VMEM((WIN,), jnp.float32),
            pltpu.VMEM((640,), jnp.float32),
            pltpu.VMEM_SHARED((N_PAD,), jnp.float32),
            pltpu.SemaphoreType.DMA,
            pltpu.SemaphoreType.DMA,
            pltpu.SemaphoreType.DMA,
            pltpu.SemaphoreType.DMA,
            pltpu.SemaphoreType.DMA,
            pltpu.SemaphoreType.DMA,
        ],
        compiler_params=_sc_params(),
    )
    def k(px_hbm, py_hbm, pz_hbm, src_hbm, dst_hbm, dist_hbm, deg_hbm,
          pxv, pyv, pzv, srcv, dstv, distv, onesv, zv, deg_sp,
          ia, ib, oa, ob, da, db_):
        cid = lax.axis_index("c")
        tid = lax.axis_index("s")
        wid = tid * NC + cid
        isem = (ia, ib)
        osem = (oa, ob)
        dsem = (da, db_)

        pltpu.sync_copy(px_hbm, pxv)
        pltpu.sync_copy(py_hbm, pyv)
        pltpu.sync_copy(pz_hbm, pzv)

        @pl.loop(0, 640 // L)
        def _(q):
            zv[pl.ds(q * L, L)] = jnp.zeros((L,), jnp.float32)

        @pl.loop(0, WIN // L)
        def _(q):
            onesv[pl.ds(q * L, L)] = jnp.full((L,), 1.0, jnp.float32)

        pltpu.sync_copy(zv, deg_sp.at[pl.ds(tid * 640, 640)])

        plsc.subcore_barrier()

        def stage(j, b):
            base = (wid + j * NW) * WIN
            pltpu.async_copy(src_hbm.at[pl.ds(base, WIN)], srcv.at[b],
                             isem[b])
            pltpu.async_copy(dst_hbm.at[pl.ds(base, WIN)], dstv.at[b],
                             isem[b])

        def wait_stage(b):
            pltpu.make_async_copy(src_hbm.at[pl.ds(0, WIN)], srcv.at[b],
                                  isem[b]).wait()
            pltpu.make_async_copy(dst_hbm.at[pl.ds(0, WIN)], dstv.at[b],
                                  isem[b]).wait()

        def compute(b):
            @plsc.parallel_loop(0, WIN // L, unroll=8)
            def _(q):
                sl = pl.ds(q * L, L)
                si = srcv[b, sl]
                di = dstv[b, sl]
                dx = plsc.load_gather(pxv, [di]) - plsc.load_gather(pxv, [si])
                dy = plsc.load_gather(pyv, [di]) - plsc.load_gather(pyv, [si])
                dz = plsc.load_gather(pzv, [di]) - plsc.load_gather(pzv, [si])
                d2 = dx * dx + dy * dy + dz * dz + 1e-12
                yy = plsc.bitcast(MAGIC - (plsc.bitcast(d2, jnp.int32) >> 1),
                                  jnp.float32)
                d2h = d2 * 0.5
                for _ in range(3):
                    yy = yy * (1.5 - d2h * yy * yy)
                distv[b, sl] = d2 * yy

        def fire_out(j, b):
            base = (wid + j * NW) * WIN
            pltpu.async_copy(distv.at[b], dist_hbm.at[pl.ds(base, WIN)],
                             osem[b])
            pltpu.async_copy(onesv, deg_sp.at[dstv.at[b]], dsem[b], add=True)

        def wait_dist_out(b):
            pltpu.make_async_copy(distv.at[b], dist_hbm.at[pl.ds(0, WIN)],
                                  osem[b]).wait()

        def wait_deg(b):
            pltpu.make_async_copy(onesv, deg_sp.at[dstv.at[b]],
                                  dsem[b]).wait()

        stage(0, 0)

        @pl.loop(0, NI, step=2)
        def _(i):
            for sb in range(2):
                j = i + sb
                b = sb
                o = 1 - sb

                @pl.when(j > 0)
                def _():
                    wait_deg(o)

                @pl.when(j < NI - 1)
                def _():
                    stage(j + 1, o)

                wait_stage(b)

                @pl.when(j > 1)
                def _():
                    wait_dist_out(b)

                compute(b)
                fire_out(j, b)

        wait_dist_out(0)
        wait_dist_out(1)
        wait_deg(1)

        plsc.subcore_barrier()

        pltpu.sync_copy(deg_sp.at[pl.ds(tid * 640, 640)], zv)
        pltpu.sync_copy(zv, deg_hbm.at[cid, pl.ds(tid * 640, 640)])

    return k(px, py, pz, src, dst)


def _sc_edge_layer(u, src, dst, dist, wd):
    @functools.partial(
        pl.kernel,
        out_type=jax.ShapeDtypeStruct((NC, N_PAD, H), jnp.float32),
        mesh=_mesh(),
        scratch_types=[
            pltpu.VMEM((2, WIN, H), jnp.float32),
            pltpu.VMEM((2, WIN), jnp.int32),
            pltpu.VMEM((2, WIN), jnp.int32),
            pltpu.VMEM((2, WIN), jnp.float32),
            pltpu.VMEM((H,), jnp.float32),
            pltpu.VMEM((ZCH, H), jnp.float32),
            pltpu.VMEM_SHARED((N_PAD, H), jnp.float32),
            pltpu.SemaphoreType.DMA,
            pltpu.SemaphoreType.DMA,
            pltpu.SemaphoreType.DMA,
            pltpu.SemaphoreType.DMA,
            pltpu.SemaphoreType.DMA,
            pltpu.SemaphoreType.DMA,
            pltpu.SemaphoreType.DMA,
            pltpu.SemaphoreType.DMA,
        ],
        compiler_params=_sc_params(),
    )
    def k(u_hbm, src_hbm, dst_hbm, dist_hbm, wd_hbm, out_hbm,
          rows, srcv, dstv, distv, wdv, zrow, s_sp,
          g0, g1, i0, i1, s0, s1, d0, d1):
        cid = lax.axis_index("c")
        tid = lax.axis_index("s")
        wid = tid * NC + cid
        gsem = (g0, g1)
        isem = (i0, i1)
        ssem = (s0, s1)
        dsem = (d0, d1)

        pltpu.sync_copy(wd_hbm, wdv)

        @pl.loop(0, ZCH)
        def _(r):
            for j in range(H // L):
                zrow[r, pl.ds(j * L, L)] = jnp.zeros((L,), jnp.float32)

        for i in range(8):
            pltpu.sync_copy(zrow, s_sp.at[pl.ds(tid * 640 + i * ZCH, ZCH)])

        plsc.subcore_barrier()

        def stage_srcdist(j, b):
            base = (wid + j * NW) * WIN
            pltpu.async_copy(src_hbm.at[pl.ds(base, WIN)], srcv.at[b], isem[b])
            pltpu.async_copy(dist_hbm.at[pl.ds(base, WIN)], distv.at[b],
                             isem[b])

        def wait_srcdist(b):
            pltpu.make_async_copy(src_hbm.at[pl.ds(0, WIN)], srcv.at[b],
                                  isem[b]).wait()
            pltpu.make_async_copy(dist_hbm.at[pl.ds(0, WIN)], distv.at[b],
                                  isem[b]).wait()

        def stage_dst(j, b):
            base = (wid + j * NW) * WIN
            pltpu.async_copy(dst_hbm.at[pl.ds(base, WIN)], dstv.at[b],
                             dsem[b])

        def wait_dst(b):
            pltpu.make_async_copy(dst_hbm.at[pl.ds(0, WIN)], dstv.at[b],
                                  dsem[b]).wait()

        def fire_gather(b):
            pltpu.async_copy(u_hbm.at[srcv.at[b]], rows.at[b], gsem[b])

        def wait_gather(b):
            pltpu.make_async_copy(u_hbm.at[srcv.at[b]], rows.at[b],
                                  gsem[b]).wait()

        def fire_scatter(b):
            pltpu.async_copy(rows.at[b], s_sp.at[dstv.at[b]], ssem[b], add=True)

        def wait_scatter(b):
            pltpu.make_async_copy(rows.at[b], s_sp.at[dstv.at[b]],
                                  ssem[b]).wait()

        def compute(b):
            @plsc.parallel_loop(0, WIN, unroll=16)
            def _(e):
                db = plsc.load_gather(distv.at[b],
                                      [jnp.full((L,), e, jnp.int32)])
                for jj in range(H // L):
                    sl = pl.ds(jj * L, L)
                    rows[b, e, sl] = jnp.maximum(rows[b, e, sl] + db * wdv[sl],
                                                 0.0)

        stage_srcdist(0, 0)
        stage_srcdist(1, 1)
        stage_dst(0, 0)
        wait_srcdist(0)
        fire_gather(0)

        @pl.loop(0, NI, step=2)
        def _(i):
            for sb in range(2):
                j = i + sb
                b = sb
                o = 1 - sb

                @pl.when(j > 0)
                def _():
                    wait_scatter(o)

                wait_gather(b)

                @pl.when(j < NI - 1)
                def _():
                    wait_srcdist(o)
                    fire_gather(o)
                    stage_dst(j + 1, o)

                compute(b)
                wait_dst(b)
                fire_scatter(b)

                @pl.when(j < NI - 2)
                def _():
                    stage_srcdist(j + 2, b)

        wait_scatter(1)

        plsc.subcore_barrier()

        for i in range(8):
            b = tid * 640 + i * ZCH
            pltpu.sync_copy(s_sp.at[pl.ds(b, ZCH)], zrow)
            pltpu.sync_copy(zrow, out_hbm.at[cid, pl.ds(b, ZCH)])

    return k(u, src, dst, dist, wd)


def _tc_lin0(h, W, b):
    def body(h_ref, w_ref, b_ref, o_ref):
        o_ref[...] = (
            jnp.dot(h_ref[...], w_ref[...], preferred_element_type=jnp.float32)
            + b_ref[...]
        )

    return pl.pallas_call(
        body,
        grid=(NB,),
        in_specs=[
            pl.BlockSpec((BN, D), lambda i: (i, 0)),
            pl.BlockSpec((D, H), lambda i: (0, 0)),
            pl.BlockSpec((1, H), lambda i: (0, 0)),
        ],
        out_specs=pl.BlockSpec((BN, H), lambda i: (i, 0)),
        out_shape=jax.ShapeDtypeStruct((N, H), jnp.float32),
    )(h, W, b)


def _tc_mid(s_p, deg_p, W2, b2, W1n, b1n):
    def body(sp_ref, dg_ref, w2_ref, b2_ref, w1_ref, b1_ref, o_ref):
        s = sp_ref[0] + sp_ref[1]
        deg = dg_ref[0] + dg_ref[1]
        hh = jnp.maximum(
            jnp.dot(s, w2_ref[...], preferred_element_type=jnp.float32)
            + deg * b2_ref[...], 0.0)
        o_ref[...] = (
            jnp.dot(hh, w1_ref[...], preferred_element_type=jnp.float32)
            + b1_ref[...]
        )

    return pl.pallas_call(
        body,
        grid=(NB,),
        in_specs=[
            pl.BlockSpec((NC, BN, H), lambda i: (0, i, 0)),
            pl.BlockSpec((NC, BN, 1), lambda i: (0, i, 0)),
            pl.BlockSpec((H, H), lambda i: (0, 0)),
            pl.BlockSpec((1, H), lambda i: (0, 0)),
            pl.BlockSpec((H, H), lambda i: (0, 0)),
            pl.BlockSpec((1, H), lambda i: (0, 0)),
        ],
        out_specs=pl.BlockSpec((BN, H), lambda i: (i, 0)),
        out_shape=jax.ShapeDtypeStruct((N, H), jnp.float32),
    )(s_p, deg_p, W2, b2, W1n, b1n)


def _tc_final(s_p, deg_p, W2, b2, batch3, l1w, l1b, l2w, l2b):
    def body(sp_ref, dg_ref, w2_ref, b2_ref, bat_ref,
             l1w_ref, l1b_ref, l2w_ref, l2b_ref, o_ref, sums, cnts):
        i = pl.program_id(0)

        @pl.when(i == 0)
        def _():
            sums[...] = jnp.zeros_like(sums)
            cnts[...] = jnp.zeros_like(cnts)

        s = sp_ref[0] + sp_ref[1]
        deg = dg_ref[0] + dg_ref[1]
        hh = jnp.maximum(
            jnp.dot(s, w2_ref[...], preferred_element_type=jnp.float32)
            + deg * b2_ref[...], 0.0)
        bat = bat_ref[0]
        oh = (lax.broadcasted_iota(jnp.int32, (G, BN), 0) == bat).astype(
            jnp.float32)
        sums[...] += jnp.dot(oh, hh, preferred_element_type=jnp.float32)
        cnts[...] += jnp.sum(oh, axis=1, keepdims=True)

        @pl.when(i == NB - 1)
        def _():
            g = sums[...] / jnp.maximum(cnts[...], 1.0)
            a = jnp.maximum(
                jnp.dot(g, l1w_ref[...], preferred_element_type=jnp.float32)
                + l1b_ref[...], 0.0)
            o_ref[...] = (
                jnp.dot(a, l2w_ref[...], preferred_element_type=jnp.float32)
                + l2b_ref[...]
            )

    return pl.pallas_call(
        body,
        grid=(NB,),
        in_specs=[
            pl.BlockSpec((NC, BN, H), lambda i: (0, i, 0)),
            pl.BlockSpec((NC, BN, 1), lambda i: (0, i, 0)),
            pl.BlockSpec((H, H), lambda i: (0, 0)),
            pl.BlockSpec((1, H), lambda i: (0, 0)),
            pl.BlockSpec((1, 1, BN), lambda i: (i, 0, 0)),
            pl.BlockSpec((H, H), lambda i: (0, 0)),
            pl.BlockSpec((1, H), lambda i: (0, 0)),
            pl.BlockSpec((H, H), lambda i: (0, 0)),
            pl.BlockSpec((1, H), lambda i: (0, 0)),
        ],
        out_specs=pl.BlockSpec((G, H), lambda i: (0, 0)),
        out_shape=jax.ShapeDtypeStruct((G, H), jnp.float32),
        scratch_shapes=[
            pltpu.VMEM((G, H), jnp.float32),
            pltpu.VMEM((G, 1), jnp.float32),
        ],
    )(s_p, deg_p, W2, b2, batch3, l1w, l1b, l2w, l2b)


def kernel(x, pos, edge_index, batch,
           c1_W1, c1_b1, c1_W2, c1_b2,
           c2_W1, c2_b1, c2_W2, c2_b2,
           lin1_W, lin1_b, lin2_W, lin2_b):
    pade = E_PAD - E
    pidx = jnp.arange(pade, dtype=jnp.int32)
    src = jnp.concatenate([edge_index[0], pidx % N])
    dst = jnp.concatenate([edge_index[1], N + pidx % (N_PAD - N)])

    dist, deg_p = _sc_preprocess(pos[:, 0], pos[:, 1], pos[:, 2], src, dst)
    deg_p3 = deg_p.reshape(NC, N_PAD, 1)

    u1 = _tc_lin0(x, c1_W1[:D], c1_b1.reshape(1, H))
    s1 = _sc_edge_layer(u1, src, dst, dist, c1_W1[D])

    u2 = _tc_mid(s1, deg_p3, c1_W2, c1_b2.reshape(1, H),
                 c2_W1[:D], c2_b1.reshape(1, H))
    s2 = _sc_edge_layer(u2, src, dst, dist, c2_W1[D])

    return _tc_final(s2, deg_p3, c2_W2, c2_b2.reshape(1, H),
                     batch.reshape(NB, 1, BN),
                     lin1_W, lin1_b.reshape(1, H),
                     lin2_W, lin2_b.reshape(1, H))


# speedup vs baseline: 1.0039x; 1.0039x over previous
import dataclasses
import functools

import jax
import jax.numpy as jnp
import numpy as np
from jax import lax
from jax.experimental import pallas as pl
from jax.experimental.pallas import tpu as pltpu
from jax.experimental.pallas import tpu_sc as plsc

N = 10000
N_PAD = 10240
E = 320000
E_PAD = 327680
D = 128
H = 128
G = 64
NC = 2
NS = 16
NW = NC * NS
L = 16
WIN = 128
NWIN = E_PAD // WIN
NI = NWIN // NW
ZCH = 80
BN = 1000
NB = N // BN
MAGIC = np.int32(0x5F3759DF)


def _mesh():
    return plsc.VectorSubcoreMesh(
        core_axis_name="c", subcore_axis_name="s", num_cores=NC, num_subcores=NS
    )


def _sc_params():
    cp = pltpu.CompilerParams()
    if "needs_layout_passes" in pltpu.CompilerParams.__dataclass_fields__:
        cp = dataclasses.replace(cp, needs_layout_passes=False)
    return cp


def _sc_preprocess(px, py, pz, src, dst):
    @functools.partial(
        pl.kernel,
        out_type=(
            jax.ShapeDtypeStruct((E_PAD,), jnp.float32),
            jax.ShapeDtypeStruct((NC, N_PAD), jnp.float32),
        ),
        mesh=_mesh(),
        scratch_types=[
            pltpu.VMEM((N,), jnp.float32),
            pltpu.VMEM((N,), jnp.float32),
            pltpu.VMEM((N,), jnp.float32),
            pltpu.VMEM((2, WIN), jnp.int32),
            pltpu.VMEM((2, WIN), jnp.int32),
            pltpu.VMEM((2, WIN), jnp.float32),
            pltpu.VMEM((WIN,), jnp.float32),
            pltpu.VMEM((640,), jnp.float32),
            pltpu.VMEM_SHARED((N_PAD,), jnp.float32),
            pltpu.SemaphoreType.DMA,
            pltpu.SemaphoreType.DMA,
            pltpu.SemaphoreType.DMA,
            pltpu.SemaphoreType.DMA,
            pltpu.SemaphoreType.DMA,
            pltpu.SemaphoreType.DMA,
        ],
        compiler_params=_sc_params(),
    )
    def k(px_hbm, py_hbm, pz_hbm, src_hbm, dst_hbm, dist_hbm, deg_hbm,
          pxv, pyv, pzv, srcv, dstv, distv, onesv, zv, deg_sp,
          ia, ib, oa, ob, da, db_):
        cid = lax.axis_index("c")
        tid = lax.axis_index("s")
        wid = tid * NC + cid
        isem = (ia, ib)
        osem = (oa, ob)
        dsem = (da, db_)

        pltpu.sync_copy(px_hbm, pxv)
        pltpu.sync_copy(py_hbm, pyv)
        pltpu.sync_copy(pz_hbm, pzv)

        @pl.loop(0, 640 // L)
        def _(q):
            zv[pl.ds(q * L, L)] = jnp.zeros((L,), jnp.float32)

        @pl.loop(0, WIN // L)
        def _(q):
            onesv[pl.ds(q * L, L)] = jnp.full((L,), 1.0, jnp.float32)

        pltpu.sync_copy(zv, deg_sp.at[pl.ds(tid * 640, 640)])

        plsc.subcore_barrier()

        def stage(j, b):
            base = (wid + j * NW) * WIN
            pltpu.async_copy(src_hbm.at[pl.ds(base, WIN)], srcv.at[b],
                             isem[b])
            pltpu.async_copy(dst_hbm.at[pl.ds(base, WIN)], dstv.at[b],
                             isem[b])

        def wait_stage(b):
            pltpu.make_async_copy(src_hbm.at[pl.ds(0, WIN)], srcv.at[b],
                                  isem[b]).wait()
            pltpu.make_async_copy(dst_hbm.at[pl.ds(0, WIN)], dstv.at[b],
                                  isem[b]).wait()

        def compute(b):
            @plsc.parallel_loop(0, WIN // L, unroll=8)
            def _(q):
                sl = pl.ds(q * L, L)
                si = srcv[b, sl]
                di = dstv[b, sl]
                dx = plsc.load_gather(pxv, [di]) - plsc.load_gather(pxv, [si])
                dy = plsc.load_gather(pyv, [di]) - plsc.load_gather(pyv, [si])
                dz = plsc.load_gather(pzv, [di]) - plsc.load_gather(pzv, [si])
                d2 = dx * dx + dy * dy + dz * dz + 1e-12
                yy = plsc.bitcast(MAGIC - (plsc.bitcast(d2, jnp.int32) >> 1),
                                  jnp.float32)
                d2h = d2 * 0.5
                for _ in range(3):
                    yy = yy * (1.5 - d2h * yy * yy)
                distv[b, sl] = d2 * yy

        def fire_out(j, b):
            base = (wid + j * NW) * WIN
            pltpu.async_copy(distv.at[b], dist_hbm.at[pl.ds(base, WIN)],
                             osem[b])
            pltpu.async_copy(onesv, deg_sp.at[dstv.at[b]], dsem[b], add=True)

        def wait_dist_out(b):
            pltpu.make_async_copy(distv.at[b], dist_hbm.at[pl.ds(0, WIN)],
                                  osem[b]).wait()

        def wait_deg(b):
            pltpu.make_async_copy(onesv, deg_sp.at[dstv.at[b]],
                                  dsem[b]).wait()

        stage(0, 0)

        @pl.loop(0, NI, step=2)
        def _(i):
            for sb in range(2):
                j = i + sb
                b = sb
                o = 1 - sb

                @pl.when(j > 0)
                def _():
                    wait_deg(o)

                @pl.when(j < NI - 1)
                def _():
                    stage(j + 1, o)

                wait_stage(b)

                @pl.when(j > 1)
                def _():
                    wait_dist_out(b)

                compute(b)
                fire_out(j, b)

        wait_dist_out(0)
        wait_dist_out(1)
        wait_deg(1)

        plsc.subcore_barrier()

        pltpu.sync_copy(deg_sp.at[pl.ds(tid * 640, 640)], zv)
        pltpu.sync_copy(zv, deg_hbm.at[cid, pl.ds(tid * 640, 640)])

    return k(px, py, pz, src, dst)


def _sc_edge_layer(u, src, dst, dist, wd):
    @functools.partial(
        pl.kernel,
        out_type=jax.ShapeDtypeStruct((NC, N_PAD, H), jnp.float32),
        mesh=_mesh(),
        scratch_types=[
            pltpu.VMEM((2, WIN, H), jnp.float32),
            pltpu.VMEM((2, WIN), jnp.int32),
            pltpu.VMEM((2, WIN), jnp.int32),
            pltpu.VMEM((2, WIN), jnp.float32),
            pltpu.VMEM((H,), jnp.float32),
            pltpu.VMEM((ZCH, H), jnp.float32),
            pltpu.VMEM_SHARED((N_PAD, H), jnp.float32),
            pltpu.SemaphoreType.DMA,
            pltpu.SemaphoreType.DMA,
            pltpu.SemaphoreType.DMA,
            pltpu.SemaphoreType.DMA,
            pltpu.SemaphoreType.DMA,
            pltpu.SemaphoreType.DMA,
            pltpu.SemaphoreType.DMA,
            pltpu.SemaphoreType.DMA,
        ],
        compiler_params=_sc_params(),
    )
    def k(u_hbm, src_hbm, dst_hbm, dist_hbm, wd_hbm, out_hbm,
          rows, srcv, dstv, distv, wdv, zrow, s_sp,
          g0, g1, i0, i1, s0, s1, d0, d1):
        cid = lax.axis_index("c")
        tid = lax.axis_index("s")
        wid = tid * NC + cid
        gsem = (g0, g1)
        isem = (i0, i1)
        ssem = (s0, s1)
        dsem = (d0, d1)

        pltpu.sync_copy(wd_hbm, wdv)

        @pl.loop(0, ZCH)
        def _(r):
            for j in range(H // L):
                zrow[r, pl.ds(j * L, L)] = jnp.zeros((L,), jnp.float32)

        for i in range(8):
            pltpu.sync_copy(zrow, s_sp.at[pl.ds(tid * 640 + i * ZCH, ZCH)])

        plsc.subcore_barrier()

        def stage_srcdist(j, b):
            base = (wid + j * NW) * WIN
            pltpu.async_copy(src_hbm.at[pl.ds(base, WIN)], srcv.at[b], isem[b])
            pltpu.async_copy(dist_hbm.at[pl.ds(base, WIN)], distv.at[b],
                             isem[b])

        def wait_srcdist(b):
            pltpu.make_async_copy(src_hbm.at[pl.ds(0, WIN)], srcv.at[b],
                                  isem[b]).wait()
            pltpu.make_async_copy(dist_hbm.at[pl.ds(0, WIN)], distv.at[b],
                                  isem[b]).wait()

        def stage_dst(j, b):
            base = (wid + j * NW) * WIN
            pltpu.async_copy(dst_hbm.at[pl.ds(base, WIN)], dstv.at[b],
                             dsem[b])

        def wait_dst(b):
            pltpu.make_async_copy(dst_hbm.at[pl.ds(0, WIN)], dstv.at[b],
                                  dsem[b]).wait()

        def fire_gather(b):
            pltpu.async_copy(u_hbm.at[srcv.at[b]], rows.at[b], gsem[b])

        def wait_gather(b):
            pltpu.make_async_copy(u_hbm.at[srcv.at[b]], rows.at[b],
                                  gsem[b]).wait()

        def fire_scatter(b):
            pltpu.async_copy(rows.at[b], s_sp.at[dstv.at[b]], ssem[b], add=True)

        def wait_scatter(b):
            pltpu.make_async_copy(rows.at[b], s_sp.at[dstv.at[b]],
                                  ssem[b]).wait()

        def compute(b):
            @plsc.parallel_loop(0, WIN, unroll=8)
            def _(e):
                db = plsc.load_gather(distv.at[b],
                                      [jnp.full((L,), e, jnp.int32)])
                for jj in range(H // L):
                    sl = pl.ds(jj * L, L)
                    rows[b, e, sl] = jnp.maximum(rows[b, e, sl] + db * wdv[sl],
                                                 0.0)

        stage_srcdist(0, 0)
        stage_srcdist(1, 1)
        stage_dst(0, 0)
        wait_srcdist(0)
        fire_gather(0)

        @pl.loop(0, NI, step=2)
        def _(i):
            for sb in range(2):
                j = i + sb
                b = sb
                o = 1 - sb

                @pl.when(j > 0)
                def _():
                    wait_scatter(o)

                wait_gather(b)

                @pl.when(j < NI - 1)
                def _():
                    wait_srcdist(o)
                    fire_gather(o)
                    stage_dst(j + 1, o)

                compute(b)
                wait_dst(b)
                fire_scatter(b)

                @pl.when(j < NI - 2)
                def _():
                    stage_srcdist(j + 2, b)

        wait_scatter(1)

        plsc.subcore_barrier()

        for i in range(8):
            b = tid * 640 + i * ZCH
            pltpu.sync_copy(s_sp.at[pl.ds(b, ZCH)], zrow)
            pltpu.sync_copy(zrow, out_hbm.at[cid, pl.ds(b, ZCH)])

    return k(u, src, dst, dist, wd)


def _tc_lin0(h, W, b):
    def body(h_ref, w_ref, b_ref, o_ref):
        o_ref[...] = (
            jnp.dot(h_ref[...], w_ref[...], preferred_element_type=jnp.float32)
            + b_ref[...]
        )

    return pl.pallas_call(
        body,
        grid=(NB,),
        in_specs=[
            pl.BlockSpec((BN, D), lambda i: (i, 0)),
            pl.BlockSpec((D, H), lambda i: (0, 0)),
            pl.BlockSpec((1, H), lambda i: (0, 0)),
        ],
        out_specs=pl.BlockSpec((BN, H), lambda i: (i, 0)),
        out_shape=jax.ShapeDtypeStruct((N, H), jnp.float32),
    )(h, W, b)


def _tc_mid(s_p, deg_p, W2, b2, W1n, b1n):
    def body(sp_ref, dg_ref, w2_ref, b2_ref, w1_ref, b1_ref, o_ref):
        s = sp_ref[0] + sp_ref[1]
        deg = dg_ref[0] + dg_ref[1]
        hh = jnp.maximum(
            jnp.dot(s, w2_ref[...], preferred_element_type=jnp.float32)
            + deg * b2_ref[...], 0.0)
        o_ref[...] = (
            jnp.dot(hh, w1_ref[...], preferred_element_type=jnp.float32)
            + b1_ref[...]
        )

    return pl.pallas_call(
        body,
        grid=(NB,),
        in_specs=[
            pl.BlockSpec((NC, BN, H), lambda i: (0, i, 0)),
            pl.BlockSpec((NC, BN, 1), lambda i: (0, i, 0)),
            pl.BlockSpec((H, H), lambda i: (0, 0)),
            pl.BlockSpec((1, H), lambda i: (0, 0)),
            pl.BlockSpec((H, H), lambda i: (0, 0)),
            pl.BlockSpec((1, H), lambda i: (0, 0)),
        ],
        out_specs=pl.BlockSpec((BN, H), lambda i: (i, 0)),
        out_shape=jax.ShapeDtypeStruct((N, H), jnp.float32),
    )(s_p, deg_p, W2, b2, W1n, b1n)


def _tc_final(s_p, deg_p, W2, b2, batch3, l1w, l1b, l2w, l2b):
    def body(sp_ref, dg_ref, w2_ref, b2_ref, bat_ref,
             l1w_ref, l1b_ref, l2w_ref, l2b_ref, o_ref, sums, cnts):
        i = pl.program_id(0)

        @pl.when(i == 0)
        def _():
            sums[...] = jnp.zeros_like(sums)
            cnts[...] = jnp.zeros_like(cnts)

        s = sp_ref[0] + sp_ref[1]
        deg = dg_ref[0] + dg_ref[1]
        hh = jnp.maximum(
            jnp.dot(s, w2_ref[...], preferred_element_type=jnp.float32)
            + deg * b2_ref[...], 0.0)
        bat = bat_ref[0]
        oh = (lax.broadcasted_iota(jnp.int32, (G, BN), 0) == bat).astype(
            jnp.float32)
        sums[...] += jnp.dot(oh, hh, preferred_element_type=jnp.float32)
        cnts[...] += jnp.sum(oh, axis=1, keepdims=True)

        @pl.when(i == NB - 1)
        def _():
            g = sums[...] / jnp.maximum(cnts[...], 1.0)
            a = jnp.maximum(
                jnp.dot(g, l1w_ref[...], preferred_element_type=jnp.float32)
                + l1b_ref[...], 0.0)
            o_ref[...] = (
                jnp.dot(a, l2w_ref[...], preferred_element_type=jnp.float32)
                + l2b_ref[...]
            )

    return pl.pallas_call(
        body,
        grid=(NB,),
        in_specs=[
            pl.BlockSpec((NC, BN, H), lambda i: (0, i, 0)),
            pl.BlockSpec((NC, BN, 1), lambda i: (0, i, 0)),
            pl.BlockSpec((H, H), lambda i: (0, 0)),
            pl.BlockSpec((1, H), lambda i: (0, 0)),
            pl.BlockSpec((1, 1, BN), lambda i: (i, 0, 0)),
            pl.BlockSpec((H, H), lambda i: (0, 0)),
            pl.BlockSpec((1, H), lambda i: (0, 0)),
            pl.BlockSpec((H, H), lambda i: (0, 0)),
            pl.BlockSpec((1, H), lambda i: (0, 0)),
        ],
        out_specs=pl.BlockSpec((G, H), lambda i: (0, 0)),
        out_shape=jax.ShapeDtypeStruct((G, H), jnp.float32),
        scratch_shapes=[
            pltpu.VMEM((G, H), jnp.float32),
            pltpu.VMEM((G, 1), jnp.float32),
        ],
    )(s_p, deg_p, W2, b2, batch3, l1w, l1b, l2w, l2b)


def kernel(x, pos, edge_index, batch,
           c1_W1, c1_b1, c1_W2, c1_b2,
           c2_W1, c2_b1, c2_W2, c2_b2,
           lin1_W, lin1_b, lin2_W, lin2_b):
    pade = E_PAD - E
    pidx = jnp.arange(pade, dtype=jnp.int32)
    src = jnp.concatenate([edge_index[0], pidx % N])
    dst = jnp.concatenate([edge_index[1], N + pidx % (N_PAD - N)])

    dist, deg_p = _sc_preprocess(pos[:, 0], pos[:, 1], pos[:, 2], src, dst)
    deg_p3 = deg_p.reshape(NC, N_PAD, 1)

    u1 = _tc_lin0(x, c1_W1[:D], c1_b1.reshape(1, H))
    s1 = _sc_edge_layer(u1, src, dst, dist, c1_W1[D])

    u2 = _tc_mid(s1, deg_p3, c1_W2, c1_b2.reshape(1, H),
                 c2_W1[:D], c2_b1.reshape(1, H))
    s2 = _sc_edge_layer(u2, src, dst, dist, c2_W1[D])

    return _tc_final(s2, deg_p3, c2_W2, c2_b2.reshape(1, H),
                     batch.reshape(NB, 1, BN),
                     lin1_W, lin1_b.reshape(1, H),
                     lin2_W, lin2_b.reshape(1, H))


# speedup vs baseline: 1.0313x; 1.0273x over previous
import dataclasses
import functools

import jax
import jax.numpy as jnp
import numpy as np
from jax import lax
from jax.experimental import pallas as pl
from jax.experimental.pallas import tpu as pltpu
from jax.experimental.pallas import tpu_sc as plsc

N = 10000
N_PAD = 10240
E = 320000
E_PAD = 327680
D = 128
H = 128
G = 64
NC = 2
NS = 16
NW = NC * NS
L = 16
WIN = 128
NWIN = E_PAD // WIN
NI = NWIN // NW
ZCH = 80
BN = 1000
NB = N // BN
MAGIC = np.int32(0x5F3759DF)


def _mesh():
    return plsc.VectorSubcoreMesh(
        core_axis_name="c", subcore_axis_name="s", num_cores=NC, num_subcores=NS
    )


def _sc_params():
    cp = pltpu.CompilerParams()
    if "needs_layout_passes" in pltpu.CompilerParams.__dataclass_fields__:
        cp = dataclasses.replace(cp, needs_layout_passes=False)
    return cp


def _sc_preprocess(px, py, pz, src, dst):
    @functools.partial(
        pl.kernel,
        out_type=(
            jax.ShapeDtypeStruct((E_PAD,), jnp.float32),
            jax.ShapeDtypeStruct((NC, N_PAD), jnp.float32),
        ),
        mesh=_mesh(),
        scratch_types=[
            pltpu.VMEM((N,), jnp.float32),
            pltpu.VMEM((N,), jnp.float32),
            pltpu.VMEM((N,), jnp.float32),
            pltpu.VMEM((2, WIN), jnp.int32),
            pltpu.VMEM((2, WIN), jnp.int32),
            pltpu.VMEM((2, WIN), jnp.float32),
            pltpu.VMEM((WIN,), jnp.float32),
            pltpu.VMEM((640,), jnp.float32),
            pltpu.VMEM_SHARED((N_PAD,), jnp.float32),
            pltpu.SemaphoreType.DMA,
            pltpu.SemaphoreType.DMA,
            pltpu.SemaphoreType.DMA,
            pltpu.SemaphoreType.DMA,
            pltpu.SemaphoreType.DMA,
            pltpu.SemaphoreType.DMA,
        ],
        compiler_params=_sc_params(),
    )
    def k(px_hbm, py_hbm, pz_hbm, src_hbm, dst_hbm, dist_hbm, deg_hbm,
          pxv, pyv, pzv, srcv, dstv, distv, onesv, zv, deg_sp,
          ia, ib, oa, ob, da, db_):
        cid = lax.axis_index("c")
        tid = lax.axis_index("s")
        wid = tid * NC + cid
        isem = (ia, ib)
        osem = (oa, ob)
        dsem = (da, db_)

        pltpu.sync_copy(px_hbm, pxv)
        pltpu.sync_copy(py_hbm, pyv)
        pltpu.sync_copy(pz_hbm, pzv)

        @pl.loop(0, 640 // L)
        def _(q):
            zv[pl.ds(q * L, L)] = jnp.zeros((L,), jnp.float32)

        @pl.loop(0, WIN // L)
        def _(q):
            onesv[pl.ds(q * L, L)] = jnp.full((L,), 1.0, jnp.float32)

        pltpu.sync_copy(zv, deg_sp.at[pl.ds(tid * 640, 640)])

        plsc.subcore_barrier()

        def stage(j, b):
            base = (wid + j * NW) * WIN
            pltpu.async_copy(src_hbm.at[pl.ds(base, WIN)], srcv.at[b],
                             isem[b])
            pltpu.async_copy(dst_hbm.at[pl.ds(base, WIN)], dstv.at[b],
                             isem[b])

        def wait_stage(b):
            pltpu.make_async_copy(src_hbm.at[pl.ds(0, WIN)], srcv.at[b],
                                  isem[b]).wait()
            pltpu.make_async_copy(dst_hbm.at[pl.ds(0, WIN)], dstv.at[b],
                                  isem[b]).wait()

        def compute(b):
            @plsc.parallel_loop(0, WIN // L, unroll=8)
            def _(q):
                sl = pl.ds(q * L, L)
                si = srcv[b, sl]
                di = dstv[b, sl]
                dx = plsc.load_gather(pxv, [di]) - plsc.load_gather(pxv, [si])
                dy = plsc.load_gather(pyv, [di]) - plsc.load_gather(pyv, [si])
                dz = plsc.load_gather(pzv, [di]) - plsc.load_gather(pzv, [si])
                d2 = dx * dx + dy * dy + dz * dz + 1e-12
                yy = plsc.bitcast(MAGIC - (plsc.bitcast(d2, jnp.int32) >> 1),
                                  jnp.float32)
                d2h = d2 * 0.5
                for _ in range(3):
                    yy = yy * (1.5 - d2h * yy * yy)
                distv[b, sl] = d2 * yy

        def fire_out(j, b):
            base = (wid + j * NW) * WIN
            pltpu.async_copy(distv.at[b], dist_hbm.at[pl.ds(base, WIN)],
                             osem[b])
            pltpu.async_copy(onesv, deg_sp.at[dstv.at[b]], dsem[b], add=True)

        def wait_dist_out(b):
            pltpu.make_async_copy(distv.at[b], dist_hbm.at[pl.ds(0, WIN)],
                                  osem[b]).wait()

        def wait_deg(b):
            pltpu.make_async_copy(onesv, deg_sp.at[dstv.at[b]],
                                  dsem[b]).wait()

        stage(0, 0)

        @pl.loop(0, NI, step=2)
        def _(i):
            for sb in range(2):
                j = i + sb
                b = sb
                o = 1 - sb

                @pl.when(j > 0)
                def _():
                    wait_deg(o)

                @pl.when(j < NI - 1)
                def _():
                    stage(j + 1, o)

                wait_stage(b)

                @pl.when(j > 1)
                def _():
                    wait_dist_out(b)

                compute(b)
                fire_out(j, b)

        wait_dist_out(0)
        wait_dist_out(1)
        wait_deg(1)

        plsc.subcore_barrier()

        pltpu.sync_copy(deg_sp.at[pl.ds(tid * 640, 640)], zv)
        pltpu.sync_copy(zv, deg_hbm.at[cid, pl.ds(tid * 640, 640)])

    return k(px, py, pz, src, dst)


def _sc_edge_layer(u, src, dst, dist, wd):
    @functools.partial(
        pl.kernel,
        out_type=jax.ShapeDtypeStruct((NC, N_PAD, H), jnp.float32),
        mesh=_mesh(),
        scratch_types=[
            pltpu.VMEM((2, WIN, H), jnp.float32),
            pltpu.VMEM((2, WIN), jnp.int32),
            pltpu.VMEM((2, WIN), jnp.int32),
            pltpu.VMEM((2, WIN), jnp.float32),
            pltpu.VMEM((H,), jnp.float32),
            pltpu.VMEM((ZCH, H), jnp.float32),
            pltpu.VMEM_SHARED((N_PAD, H), jnp.float32),
            pltpu.SemaphoreType.DMA,
            pltpu.SemaphoreType.DMA,
            pltpu.SemaphoreType.DMA,
            pltpu.SemaphoreType.DMA,
            pltpu.SemaphoreType.DMA,
            pltpu.SemaphoreType.DMA,
            pltpu.SemaphoreType.DMA,
            pltpu.SemaphoreType.DMA,
        ],
        compiler_params=_sc_params(),
    )
    def k(u_hbm, src_hbm, dst_hbm, dist_hbm, wd_hbm, out_hbm,
          rows, srcv, dstv, distv, wdv, zrow, s_sp,
          g0, g1, i0, i1, s0, s1, d0, d1):
        cid = lax.axis_index("c")
        tid = lax.axis_index("s")
        wid = tid * NC + cid
        gsem = (g0, g1)
        isem = (i0, i1)
        ssem = (s0, s1)
        dsem = (d0, d1)

        pltpu.sync_copy(wd_hbm, wdv)

        @pl.loop(0, ZCH)
        def _(r):
            for j in range(H // L):
                zrow[r, pl.ds(j * L, L)] = jnp.zeros((L,), jnp.float32)

        for i in range(8):
            pltpu.sync_copy(zrow, s_sp.at[pl.ds(tid * 640 + i * ZCH, ZCH)])

        plsc.subcore_barrier()

        def stage_srcdist(j, b):
            base = (wid + j * NW) * WIN
            pltpu.async_copy(src_hbm.at[pl.ds(base, WIN)], srcv.at[b], isem[b])
            pltpu.async_copy(dist_hbm.at[pl.ds(base, WIN)], distv.at[b],
                             isem[b])

        def wait_srcdist(b):
            pltpu.make_async_copy(src_hbm.at[pl.ds(0, WIN)], srcv.at[b],
                                  isem[b]).wait()
            pltpu.make_async_copy(dist_hbm.at[pl.ds(0, WIN)], distv.at[b],
                                  isem[b]).wait()

        def stage_dst(j, b):
            base = (wid + j * NW) * WIN
            pltpu.async_copy(dst_hbm.at[pl.ds(base, WIN)], dstv.at[b],
                             dsem[b])

        def wait_dst(b):
            pltpu.make_async_copy(dst_hbm.at[pl.ds(0, WIN)], dstv.at[b],
                                  dsem[b]).wait()

        def fire_gather(b):
            pltpu.async_copy(u_hbm.at[srcv.at[b]], rows.at[b], gsem[b])

        def wait_gather(b):
            pltpu.make_async_copy(u_hbm.at[srcv.at[b]], rows.at[b],
                                  gsem[b]).wait()

        def fire_scatter(b):
            pltpu.async_copy(rows.at[b], s_sp.at[dstv.at[b]], ssem[b], add=True)

        def wait_scatter(b):
            pltpu.make_async_copy(rows.at[b], s_sp.at[dstv.at[b]],
                                  ssem[b]).wait()

        def compute(b):
            @plsc.parallel_loop(0, WIN, unroll=4)
            def _(e):
                db = plsc.load_gather(distv.at[b],
                                      [jnp.full((L,), e, jnp.int32)])
                for jj in range(H // L):
                    sl = pl.ds(jj * L, L)
                    rows[b, e, sl] = jnp.maximum(rows[b, e, sl] + db * wdv[sl],
                                                 0.0)

        stage_srcdist(0, 0)
        stage_srcdist(1, 1)
        stage_dst(0, 0)
        wait_srcdist(0)
        fire_gather(0)

        @pl.loop(0, NI, step=2)
        def _(i):
            for sb in range(2):
                j = i + sb
                b = sb
                o = 1 - sb

                @pl.when(j > 0)
                def _():
                    wait_scatter(o)

                wait_gather(b)

                @pl.when(j < NI - 1)
                def _():
                    wait_srcdist(o)
                    fire_gather(o)
                    stage_dst(j + 1, o)

                compute(b)
                wait_dst(b)
                fire_scatter(b)

                @pl.when(j < NI - 2)
                def _():
                    stage_srcdist(j + 2, b)

        wait_scatter(1)

        plsc.subcore_barrier()

        for i in range(8):
            b = tid * 640 + i * ZCH
            pltpu.sync_copy(s_sp.at[pl.ds(b, ZCH)], zrow)
            pltpu.sync_copy(zrow, out_hbm.at[cid, pl.ds(b, ZCH)])

    return k(u, src, dst, dist, wd)


def _tc_lin0(h, W, b):
    def body(h_ref, w_ref, b_ref, o_ref):
        o_ref[...] = (
            jnp.dot(h_ref[...], w_ref[...], preferred_element_type=jnp.float32)
            + b_ref[...]
        )

    return pl.pallas_call(
        body,
        grid=(NB,),
        in_specs=[
            pl.BlockSpec((BN, D), lambda i: (i, 0)),
            pl.BlockSpec((D, H), lambda i: (0, 0)),
            pl.BlockSpec((1, H), lambda i: (0, 0)),
        ],
        out_specs=pl.BlockSpec((BN, H), lambda i: (i, 0)),
        out_shape=jax.ShapeDtypeStruct((N, H), jnp.float32),
    )(h, W, b)


def _tc_mid(s_p, deg_p, W2, b2, W1n, b1n):
    def body(sp_ref, dg_ref, w2_ref, b2_ref, w1_ref, b1_ref, o_ref):
        s = sp_ref[0] + sp_ref[1]
        deg = dg_ref[0] + dg_ref[1]
        hh = jnp.maximum(
            jnp.dot(s, w2_ref[...], preferred_element_type=jnp.float32)
            + deg * b2_ref[...], 0.0)
        o_ref[...] = (
            jnp.dot(hh, w1_ref[...], preferred_element_type=jnp.float32)
            + b1_ref[...]
        )

    return pl.pallas_call(
        body,
        grid=(NB,),
        in_specs=[
            pl.BlockSpec((NC, BN, H), lambda i: (0, i, 0)),
            pl.BlockSpec((NC, BN, 1), lambda i: (0, i, 0)),
            pl.BlockSpec((H, H), lambda i: (0, 0)),
            pl.BlockSpec((1, H), lambda i: (0, 0)),
            pl.BlockSpec((H, H), lambda i: (0, 0)),
            pl.BlockSpec((1, H), lambda i: (0, 0)),
        ],
        out_specs=pl.BlockSpec((BN, H), lambda i: (i, 0)),
        out_shape=jax.ShapeDtypeStruct((N, H), jnp.float32),
    )(s_p, deg_p, W2, b2, W1n, b1n)


def _tc_final(s_p, deg_p, W2, b2, batch3, l1w, l1b, l2w, l2b):
    def body(sp_ref, dg_ref, w2_ref, b2_ref, bat_ref,
             l1w_ref, l1b_ref, l2w_ref, l2b_ref, o_ref, sums, cnts):
        i = pl.program_id(0)

        @pl.when(i == 0)
        def _():
            sums[...] = jnp.zeros_like(sums)
            cnts[...] = jnp.zeros_like(cnts)

        s = sp_ref[0] + sp_ref[1]
        deg = dg_ref[0] + dg_ref[1]
        hh = jnp.maximum(
            jnp.dot(s, w2_ref[...], preferred_element_type=jnp.float32)
            + deg * b2_ref[...], 0.0)
        bat = bat_ref[0]
        oh = (lax.broadcasted_iota(jnp.int32, (G, BN), 0) == bat).astype(
            jnp.float32)
        sums[...] += jnp.dot(oh, hh, preferred_element_type=jnp.float32)
        cnts[...] += jnp.sum(oh, axis=1, keepdims=True)

        @pl.when(i == NB - 1)
        def _():
            g = sums[...] / jnp.maximum(cnts[...], 1.0)
            a = jnp.maximum(
                jnp.dot(g, l1w_ref[...], preferred_element_type=jnp.float32)
                + l1b_ref[...], 0.0)
            o_ref[...] = (
                jnp.dot(a, l2w_ref[...], preferred_element_type=jnp.float32)
                + l2b_ref[...]
            )

    return pl.pallas_call(
        body,
        grid=(NB,),
        in_specs=[
            pl.BlockSpec((NC, BN, H), lambda i: (0, i, 0)),
            pl.BlockSpec((NC, BN, 1), lambda i: (0, i, 0)),
            pl.BlockSpec((H, H), lambda i: (0, 0)),
            pl.BlockSpec((1, H), lambda i: (0, 0)),
            pl.BlockSpec((1, 1, BN), lambda i: (i, 0, 0)),
            pl.BlockSpec((H, H), lambda i: (0, 0)),
            pl.BlockSpec((1, H), lambda i: (0, 0)),
            pl.BlockSpec((H, H), lambda i: (0, 0)),
            pl.BlockSpec((1, H), lambda i: (0, 0)),
        ],
        out_specs=pl.BlockSpec((G, H), lambda i: (0, 0)),
        out_shape=jax.ShapeDtypeStruct((G, H), jnp.float32),
        scratch_shapes=[
            pltpu.VMEM((G, H), jnp.float32),
            pltpu.VMEM((G, 1), jnp.float32),
        ],
    )(s_p, deg_p, W2, b2, batch3, l1w, l1b, l2w, l2b)


def kernel(x, pos, edge_index, batch,
           c1_W1, c1_b1, c1_W2, c1_b2,
           c2_W1, c2_b1, c2_W2, c2_b2,
           lin1_W, lin1_b, lin2_W, lin2_b):
    pade = E_PAD - E
    pidx = jnp.arange(pade, dtype=jnp.int32)
    src = jnp.concatenate([edge_index[0], pidx % N])
    dst = jnp.concatenate([edge_index[1], N + pidx % (N_PAD - N)])

    dist, deg_p = _sc_preprocess(pos[:, 0], pos[:, 1], pos[:, 2], src, dst)
    deg_p3 = deg_p.reshape(NC, N_PAD, 1)

    u1 = _tc_lin0(x, c1_W1[:D], c1_b1.reshape(1, H))
    s1 = _sc_edge_layer(u1, src, dst, dist, c1_W1[D])

    u2 = _tc_mid(s1, deg_p3, c1_W2, c1_b2.reshape(1, H),
                 c2_W1[:D], c2_b1.reshape(1, H))
    s2 = _sc_edge_layer(u2, src, dst, dist, c2_W1[D])

    return _tc_final(s2, deg_p3, c2_W2, c2_b2.reshape(1, H),
                     batch.reshape(NB, 1, BN),
                     lin1_W, lin1_b.reshape(1, H),
                     lin2_W, lin2_b.reshape(1, H))
